# baseline (speedup 1.0000x reference)
import functools

import jax
import jax.numpy as jnp
from jax import lax
from jax.experimental import pallas as pl
from jax.experimental.pallas import tpu as pltpu
from jax.experimental.pallas import tpu_sc as plsc

INDEX_IN_SEQ = 0


def kernel(x):
    B, S, D = x.shape
    n_workers = 32
    chunks_per_row = n_workers // B
    chunk = D // chunks_per_row

    mesh = plsc.VectorSubcoreMesh(core_axis_name="c", subcore_axis_name="s")
    rows = x.reshape(B * S, D)

    @functools.partial(
        pl.kernel,
        out_type=jax.ShapeDtypeStruct((B, D), x.dtype),
        mesh=mesh,
        scratch_types=[pltpu.VMEM((1, chunk), jnp.float32)],
    )
    def extract(rows_hbm, out_hbm, buf):
        wid = lax.axis_index("s") * 2 + lax.axis_index("c")
        b = wid // chunks_per_row
        off = (wid % chunks_per_row) * chunk
        src_row = b * S + INDEX_IN_SEQ
        pltpu.sync_copy(rows_hbm.at[pl.ds(src_row, 1), pl.ds(off, chunk)], buf)
        pltpu.sync_copy(buf, out_hbm.at[pl.ds(b, 1), pl.ds(off, chunk)])

    return extract(rows)


# speedup vs baseline: 1.0096x; 1.0096x over previous
import functools

import jax
import jax.numpy as jnp
from jax import lax
from jax.experimental import pallas as pl
from jax.experimental.pallas import tpu as pltpu
from jax.experimental.pallas import tpu_sc as plsc

INDEX_IN_SEQ = 0


def kernel(x):
    B, S, D = x.shape
    mesh = plsc.ScalarSubcoreMesh(axis_name="c", num_cores=2)
    rows = x.reshape(B * S, D)

    @functools.partial(
        pl.kernel,
        out_type=jax.ShapeDtypeStruct((B, D), x.dtype),
        mesh=mesh,
    )
    def extract(rows_hbm, out_hbm):
        c = lax.axis_index("c")
        for i in range(B // 2):
            b = c * (B // 2) + i
            src_row = b * S + INDEX_IN_SEQ
            pltpu.sync_copy(rows_hbm.at[pl.ds(src_row, 1), :],
                            out_hbm.at[pl.ds(b, 1), :])

    return extract(rows)
